# baseline (speedup 1.0000x reference)
import functools

import jax
import jax.numpy as jnp
from jax import lax
from jax.experimental import pallas as pl
from jax.experimental.pallas import tpu as pltpu
from jax.experimental.pallas import tpu_sc as plsc

NS = 16
CHUNK = 128
ZROWS = 8
DEGW = 16
NBUF = 4


def _sc_aggregate(xa, xb, src3, dst3, n_pad, nj):
    n, fh = xa.shape
    rows_out = n_pad // NS
    zch = n_pad // (NS * ZROWS)

    mesh = plsc.VectorSubcoreMesh(core_axis_name="c", subcore_axis_name="s")

    @functools.partial(
        pl.kernel,
        mesh=mesh,
        compiler_params=pltpu.CompilerParams(use_tc_tiling_on_sc=False),
        out_type=[
            jax.ShapeDtypeStruct((2, n_pad, fh), jnp.float32),
            jax.ShapeDtypeStruct((2, n_pad, DEGW), jnp.float32),
        ],
        scratch_types=[
            pltpu.VMEM((nj + 2, CHUNK), jnp.int32),
            pltpu.VMEM((nj + 2, CHUNK), jnp.int32),
            [pltpu.VMEM((CHUNK, fh), jnp.float32) for _ in range(NBUF)],
            pltpu.VMEM((CHUNK, DEGW), jnp.float32),
            pltpu.VMEM((ZROWS, fh), jnp.float32),
            pltpu.VMEM((ZROWS, DEGW), jnp.float32),
            pltpu.VMEM_SHARED((n_pad, fh), jnp.float32),
            pltpu.VMEM_SHARED((n_pad, DEGW), jnp.float32),
            [pltpu.SemaphoreType.DMA for _ in range(NBUF)],
            [pltpu.SemaphoreType.DMA for _ in range(NBUF)],
            pltpu.SemaphoreType.DMA,
        ],
    )
    def agg_kernel(xa_hbm, xb_hbm, src_hbm, dst_hbm, agg_out, deg_out,
                   src_v, dst_v, rows, ones_v, zb_v, zbd_v, agg_sh, deg_sh,
                   gsem, ssem, dsem):
        c = lax.axis_index("c")
        s = lax.axis_index("s")

        def fill_row(i, carry):
            for k in range(fh // 16):
                zb_v[i, pl.ds(k * 16, 16)] = jnp.zeros((16,), jnp.float32)
            zbd_v[i, :] = jnp.zeros((16,), jnp.float32)
            return carry
        lax.fori_loop(0, ZROWS, fill_row, 0)

        def fill_ones(i, carry):
            ones_v[i, :] = jnp.ones((16,), jnp.float32)
            return carry
        lax.fori_loop(0, CHUNK, fill_ones, 0)

        def zero_chunk(t, carry):
            row0 = s * (zch * ZROWS) + t * ZROWS
            pltpu.sync_copy(zb_v, agg_sh.at[pl.ds(row0, ZROWS)])
            pltpu.sync_copy(zbd_v, deg_sh.at[pl.ds(row0, ZROWS)])
            return carry
        lax.fori_loop(0, zch, zero_chunk, 0)

        plsc.subcore_barrier()

        pltpu.sync_copy(src_hbm.at[s], src_v)
        pltpu.sync_copy(dst_hbm.at[s], dst_v)

        def g_start(j, b):
            @pl.when(c == 0)
            def _():
                pltpu.async_copy(xa_hbm.at[src_v.at[j]], rows[b], gsem[b])

            @pl.when(c == 1)
            def _():
                pltpu.async_copy(xb_hbm.at[src_v.at[j]], rows[b], gsem[b])

        def g_wait(j, b):
            @pl.when(c == 0)
            def _():
                pltpu.make_async_copy(
                    xa_hbm.at[src_v.at[j]], rows[b], gsem[b]).wait()

            @pl.when(c == 1)
            def _():
                pltpu.make_async_copy(
                    xb_hbm.at[src_v.at[j]], rows[b], gsem[b]).wait()

        def s_start(j, b):
            pltpu.async_copy(rows[b], agg_sh.at[dst_v.at[j]], ssem[b],
                             add=True)

        def s_wait(j, b):
            pltpu.make_async_copy(rows[b], agg_sh.at[dst_v.at[j]],
                                  ssem[b]).wait()

        def d_start(j):
            pltpu.async_copy(ones_v, deg_sh.at[dst_v.at[j]], dsem, add=True)

        def d_wait(j):
            pltpu.make_async_copy(ones_v, deg_sh.at[dst_v.at[j]],
                                  dsem).wait()

        g_start(0, 0)
        g_start(1, 1)

        g_start(2, 2)
        g_wait(0, 0)
        s_start(0, 0)

        @pl.when(c == 0)
        def _():
            d_start(0)

        g_start(3, 3)
        g_wait(1, 1)
        s_start(1, 1)

        @pl.when(c == 1)
        def _():
            d_start(1)

        s_wait(0, 0)
        g_start(4, 0)
        g_wait(2, 2)
        s_start(2, 2)

        @pl.when(c == 0)
        def _():
            d_wait(0)
            d_start(2)

        s_wait(1, 1)
        g_start(5, 1)
        g_wait(3, 3)
        s_start(3, 3)

        @pl.when(c == 1)
        def _():
            d_wait(1)
            d_start(3)

        def group(j2, carry):
            for b in range(NBUF):
                j = j2 * NBUF + b
                nb = (b + 2) % NBUF
                s_wait(j - 2, nb)
                g_start(j + 2, nb)
                g_wait(j, b)
                s_start(j, b)

                @pl.when(j % 2 == c)
                def _():
                    d_wait(j - 2)
                    d_start(j)
            return carry
        lax.fori_loop(1, nj // NBUF, group, 0)

        s_wait(nj - 2, 2)
        s_wait(nj - 1, 3)
        g_wait(nj, 0)
        g_wait(nj + 1, 1)
        d_wait(nj - 2)

        plsc.subcore_barrier()

        row0 = s * rows_out
        pltpu.sync_copy(agg_sh.at[pl.ds(row0, rows_out)],
                        agg_out.at[c, pl.ds(row0, rows_out)])
        pltpu.sync_copy(deg_sh.at[pl.ds(row0, rows_out)],
                        deg_out.at[c, pl.ds(row0, rows_out)])

    return agg_kernel(xa, xb, src3, dst3)


def _mlp(x, agg2, deg2, W1, b1, W2, b2, W3, b3):
    n, f = x.shape
    fh = f // 2
    h1 = W1.shape[1]
    h2 = W2.shape[1]
    cc = W3.shape[1]
    bm = 2000
    grid = (n // bm,)

    def body(x_ref, a_ref, d_ref, w1_ref, b1_ref, w2_ref, b2_ref,
             w3_ref, b3_ref, o_ref):
        xb = x_ref[...]
        a = jnp.concatenate([a_ref[0], a_ref[1]], axis=-1)
        d = d_ref[0, :, 0:1] + d_ref[1, :, 0:1]
        a = a / jnp.maximum(d, 1.0)
        w1 = w1_ref[...]
        h = (jnp.dot(xb, w1[0:f], preferred_element_type=jnp.float32)
             + jnp.dot(a, w1[f:2 * f], preferred_element_type=jnp.float32)
             + b1_ref[...])
        h = jnp.maximum(h, 0.0)
        h = jnp.dot(h, w2_ref[...], preferred_element_type=jnp.float32) + b2_ref[...]
        h = jnp.maximum(h, 0.0)
        o_ref[...] = (jnp.dot(h, w3_ref[...], preferred_element_type=jnp.float32)
                      + b3_ref[...])

    return pl.pallas_call(
        body,
        grid=grid,
        in_specs=[
            pl.BlockSpec((bm, f), lambda i: (i, 0)),
            pl.BlockSpec((2, bm, fh), lambda i: (0, i, 0)),
            pl.BlockSpec((2, bm, DEGW), lambda i: (0, i, 0)),
            pl.BlockSpec((2 * f, h1), lambda i: (0, 0)),
            pl.BlockSpec((1, h1), lambda i: (0, 0)),
            pl.BlockSpec((h1, h2), lambda i: (0, 0)),
            pl.BlockSpec((1, h2), lambda i: (0, 0)),
            pl.BlockSpec((h2, cc), lambda i: (0, 0)),
            pl.BlockSpec((1, cc), lambda i: (0, 0)),
        ],
        out_specs=pl.BlockSpec((bm, cc), lambda i: (i, 0)),
        out_shape=jax.ShapeDtypeStruct((n, cc), jnp.float32),
    )(x, agg2, deg2, W1, b1, W2, b2, W3, b3)


def kernel(inputs, neighbor, W1, b1, W2, b2, W3, b3):
    x = inputs[:, 0, :]
    n, f = x.shape
    fh = f // 2
    xa = x[:, :fh]
    xb = x[:, fh:]
    src = neighbor[0]
    dst = neighbor[1]
    e = src.shape[0]

    nj = -(-e // (NS * CHUNK))
    nj = -(-nj // NBUF) * NBUF
    e_pad = NS * nj * CHUNK
    if e_pad > e:
        src = jnp.concatenate([src, jnp.zeros((e_pad - e,), jnp.int32)])
        dst = jnp.concatenate([dst, jnp.full((e_pad - e,), n, jnp.int32)])
    src3 = src.reshape(NS, nj, CHUNK)
    dst3 = dst.reshape(NS, nj, CHUNK)
    src3 = jnp.concatenate(
        [src3, jnp.zeros((NS, 2, CHUNK), jnp.int32)], axis=1)
    dst3 = jnp.concatenate(
        [dst3, jnp.full((NS, 2, CHUNK), n, jnp.int32)], axis=1)

    n_pad = -(-(n + 1) // (NS * ZROWS)) * (NS * ZROWS)

    agg2, deg2 = _sc_aggregate(xa, xb, src3, dst3, n_pad, nj)
    return _mlp(x, agg2, deg2, W1, b1.reshape(1, -1), W2, b2.reshape(1, -1),
                W3, b3.reshape(1, -1))


# speedup vs baseline: 1.3775x; 1.3775x over previous
import functools

import jax
import jax.numpy as jnp
from jax import lax
from jax.experimental import pallas as pl
from jax.experimental.pallas import tpu as pltpu
from jax.experimental.pallas import tpu_sc as plsc

NS = 16
CHUNK = 128
ZROWS = 8
DEGW = 16
NBUF = 2


def _sc_aggregate(xa, xb, src3, dst3, n_pad, nj):
    n, fh = xa.shape
    rows_out = n_pad // NS
    zch = n_pad // (NS * ZROWS)

    mesh = plsc.VectorSubcoreMesh(core_axis_name="c", subcore_axis_name="s")

    @functools.partial(
        pl.kernel,
        mesh=mesh,
        compiler_params=pltpu.CompilerParams(use_tc_tiling_on_sc=False),
        out_type=[
            jax.ShapeDtypeStruct((2, n_pad, fh), jnp.float32),
            jax.ShapeDtypeStruct((2, n_pad, DEGW), jnp.float32),
        ],
        scratch_types=[
            pltpu.VMEM((nj + 2, CHUNK), jnp.int32),
            pltpu.VMEM((nj + 2, CHUNK), jnp.int32),
            [pltpu.VMEM((CHUNK, fh), jnp.float32) for _ in range(NBUF)],
            pltpu.VMEM((CHUNK, DEGW), jnp.float32),
            pltpu.VMEM((ZROWS, fh), jnp.float32),
            pltpu.VMEM((ZROWS, DEGW), jnp.float32),
            pltpu.VMEM_SHARED((n_pad, fh), jnp.float32),
            pltpu.VMEM_SHARED((n_pad, DEGW), jnp.float32),
            [pltpu.SemaphoreType.DMA for _ in range(NBUF)],
        ],
    )
    def agg_kernel(xa_hbm, xb_hbm, src_hbm, dst_hbm, agg_out, deg_out,
                   src_v, dst_v, rows, ones_v, zb_v, zbd_v, agg_sh, deg_sh,
                   gsem):
        c = lax.axis_index("c")
        s = lax.axis_index("s")

        def fill_row(i, carry):
            for k in range(fh // 16):
                zb_v[i, pl.ds(k * 16, 16)] = jnp.zeros((16,), jnp.float32)
            zbd_v[i, :] = jnp.zeros((16,), jnp.float32)
            return carry
        lax.fori_loop(0, ZROWS, fill_row, 0)

        def fill_ones(i, carry):
            ones_v[i, :] = jnp.ones((16,), jnp.float32)
            return carry
        lax.fori_loop(0, CHUNK, fill_ones, 0)

        def zero_chunk(t, carry):
            row0 = s * (zch * ZROWS) + t * ZROWS
            pltpu.sync_copy(zb_v, agg_sh.at[pl.ds(row0, ZROWS)])
            pltpu.sync_copy(zbd_v, deg_sh.at[pl.ds(row0, ZROWS)])
            return carry
        lax.fori_loop(0, zch, zero_chunk, 0)

        plsc.subcore_barrier()

        pltpu.sync_copy(src_hbm.at[s], src_v)
        pltpu.sync_copy(dst_hbm.at[s], dst_v)

        def g_start(j, b):
            @pl.when(c == 0)
            def _():
                pltpu.async_copy(xa_hbm.at[src_v.at[j]], rows[b], gsem[b])

            @pl.when(c == 1)
            def _():
                pltpu.async_copy(xb_hbm.at[src_v.at[j]], rows[b], gsem[b])

        def g_wait(j, b):
            @pl.when(c == 0)
            def _():
                pltpu.make_async_copy(
                    xa_hbm.at[src_v.at[j]], rows[b], gsem[b]).wait()

            @pl.when(c == 1)
            def _():
                pltpu.make_async_copy(
                    xb_hbm.at[src_v.at[j]], rows[b], gsem[b]).wait()

        g_start(0, 0)

        def group(j2, carry):
            for b in range(2):
                j = j2 * 2 + b
                g_start(j + 1, 1 - b)
                g_wait(j, b)
                pltpu.sync_copy(rows[b], agg_sh.at[dst_v.at[j]], add=True)

                @pl.when(j % 2 == c)
                def _():
                    pltpu.sync_copy(ones_v, deg_sh.at[dst_v.at[j]],
                                    add=True)
            return carry
        lax.fori_loop(0, nj // 2, group, 0)

        g_wait(nj, 0)

        plsc.subcore_barrier()

        row0 = s * rows_out
        pltpu.sync_copy(agg_sh.at[pl.ds(row0, rows_out)],
                        agg_out.at[c, pl.ds(row0, rows_out)])
        pltpu.sync_copy(deg_sh.at[pl.ds(row0, rows_out)],
                        deg_out.at[c, pl.ds(row0, rows_out)])

    return agg_kernel(xa, xb, src3, dst3)


def _mlp(x, agg2, deg2, W1, b1, W2, b2, W3, b3):
    n, f = x.shape
    fh = f // 2
    h1 = W1.shape[1]
    h2 = W2.shape[1]
    cc = W3.shape[1]
    bm = 2000
    grid = (n // bm,)

    def body(x_ref, a_ref, d_ref, w1_ref, b1_ref, w2_ref, b2_ref,
             w3_ref, b3_ref, o_ref):
        xb = x_ref[...]
        a = jnp.concatenate([a_ref[0], a_ref[1]], axis=-1)
        d = d_ref[0, :, 0:1] + d_ref[1, :, 0:1]
        a = a / jnp.maximum(d, 1.0)
        w1 = w1_ref[...]
        h = (jnp.dot(xb, w1[0:f], preferred_element_type=jnp.float32)
             + jnp.dot(a, w1[f:2 * f], preferred_element_type=jnp.float32)
             + b1_ref[...])
        h = jnp.maximum(h, 0.0)
        h = jnp.dot(h, w2_ref[...], preferred_element_type=jnp.float32) + b2_ref[...]
        h = jnp.maximum(h, 0.0)
        o_ref[...] = (jnp.dot(h, w3_ref[...], preferred_element_type=jnp.float32)
                      + b3_ref[...])

    return pl.pallas_call(
        body,
        grid=grid,
        in_specs=[
            pl.BlockSpec((bm, f), lambda i: (i, 0)),
            pl.BlockSpec((2, bm, fh), lambda i: (0, i, 0)),
            pl.BlockSpec((2, bm, DEGW), lambda i: (0, i, 0)),
            pl.BlockSpec((2 * f, h1), lambda i: (0, 0)),
            pl.BlockSpec((1, h1), lambda i: (0, 0)),
            pl.BlockSpec((h1, h2), lambda i: (0, 0)),
            pl.BlockSpec((1, h2), lambda i: (0, 0)),
            pl.BlockSpec((h2, cc), lambda i: (0, 0)),
            pl.BlockSpec((1, cc), lambda i: (0, 0)),
        ],
        out_specs=pl.BlockSpec((bm, cc), lambda i: (i, 0)),
        out_shape=jax.ShapeDtypeStruct((n, cc), jnp.float32),
    )(x, agg2, deg2, W1, b1, W2, b2, W3, b3)


def kernel(inputs, neighbor, W1, b1, W2, b2, W3, b3):
    x = inputs[:, 0, :]
    n, f = x.shape
    fh = f // 2
    xa = x[:, :fh]
    xb = x[:, fh:]
    src = neighbor[0]
    dst = neighbor[1]
    e = src.shape[0]

    nj = -(-e // (NS * CHUNK))
    nj = -(-nj // NBUF) * NBUF
    e_pad = NS * nj * CHUNK
    if e_pad > e:
        src = jnp.concatenate([src, jnp.zeros((e_pad - e,), jnp.int32)])
        dst = jnp.concatenate([dst, jnp.full((e_pad - e,), n, jnp.int32)])
    src3 = src.reshape(NS, nj, CHUNK)
    dst3 = dst.reshape(NS, nj, CHUNK)
    src3 = jnp.concatenate(
        [src3, jnp.zeros((NS, 2, CHUNK), jnp.int32)], axis=1)
    dst3 = jnp.concatenate(
        [dst3, jnp.full((NS, 2, CHUNK), n, jnp.int32)], axis=1)

    n_pad = -(-(n + 1) // (NS * ZROWS)) * (NS * ZROWS)

    agg2, deg2 = _sc_aggregate(xa, xb, src3, dst3, n_pad, nj)
    return _mlp(x, agg2, deg2, W1, b1.reshape(1, -1), W2, b2.reshape(1, -1),
                W3, b3.reshape(1, -1))


# speedup vs baseline: 1.9476x; 1.4139x over previous
import functools

import jax
import jax.numpy as jnp
from jax import lax
from jax.experimental import pallas as pl
from jax.experimental.pallas import tpu as pltpu
from jax.experimental.pallas import tpu_sc as plsc

NS = 16
CHUNK = 128
ZROWS = 8
DEGW = 16


def _sc_aggregate(xa, xb, src3, dst3, tail_src, tail_dst, n, n_pad, nj, tail):
    fh = xa.shape[1]
    rows_out = n_pad // NS
    zch = n_pad // (NS * ZROWS)

    mesh = plsc.VectorSubcoreMesh(core_axis_name="c", subcore_axis_name="s")

    @functools.partial(
        pl.kernel,
        mesh=mesh,
        compiler_params=pltpu.CompilerParams(use_tc_tiling_on_sc=False),
        out_type=[
            jax.ShapeDtypeStruct((2, n_pad, fh), jnp.float32),
            jax.ShapeDtypeStruct((2, n_pad, DEGW), jnp.float32),
        ],
        scratch_types=[
            pltpu.VMEM((nj + 1, CHUNK), jnp.int32),
            pltpu.VMEM((nj + 1, CHUNK), jnp.int32),
            [pltpu.VMEM((CHUNK, fh), jnp.float32) for _ in range(2)],
            pltpu.VMEM((CHUNK, DEGW), jnp.float32),
            pltpu.VMEM((ZROWS, fh), jnp.float32),
            pltpu.VMEM((ZROWS, DEGW), jnp.float32),
            pltpu.VMEM_SHARED((n_pad, fh), jnp.float32),
            pltpu.VMEM_SHARED((n_pad, DEGW), jnp.float32),
            [pltpu.SemaphoreType.DMA for _ in range(2)],
        ],
    )
    def agg_kernel(xa_hbm, xb_hbm, src_hbm, dst_hbm, tsrc_hbm, tdst_hbm,
                   agg_out, deg_out,
                   src_v, dst_v, rows, ones_v, zb_v, zbd_v, agg_sh, deg_sh,
                   gsem):
        c = lax.axis_index("c")
        s = lax.axis_index("s")

        def fill_row(i, carry):
            for k in range(fh // 16):
                zb_v[i, pl.ds(k * 16, 16)] = jnp.zeros((16,), jnp.float32)
            zbd_v[i, :] = jnp.zeros((16,), jnp.float32)
            return carry
        lax.fori_loop(0, ZROWS, fill_row, 0)

        def fill_ones(i, carry):
            ones_v[i, :] = jnp.ones((16,), jnp.float32)
            return carry
        lax.fori_loop(0, CHUNK, fill_ones, 0)

        def zero_chunk(t, carry):
            row0 = s * (zch * ZROWS) + t * ZROWS
            pltpu.sync_copy(zb_v, agg_sh.at[pl.ds(row0, ZROWS)])
            pltpu.sync_copy(zbd_v, deg_sh.at[pl.ds(row0, ZROWS)])
            return carry
        lax.fori_loop(0, zch, zero_chunk, 0)

        plsc.subcore_barrier()

        pltpu.sync_copy(src_hbm.at[s], src_v.at[pl.ds(0, nj)])
        pltpu.sync_copy(dst_hbm.at[s], dst_v.at[pl.ds(0, nj)])
        for k in range(CHUNK // 16):
            src_v[nj, pl.ds(k * 16, 16)] = jnp.zeros((16,), jnp.int32)
            dst_v[nj, pl.ds(k * 16, 16)] = jnp.full((16,), n, jnp.int32)
        if tail:
            @pl.when(s < tail)
            def _():
                pltpu.sync_copy(tsrc_hbm.at[s], src_v.at[pl.ds(nj, 1)])
                pltpu.sync_copy(tdst_hbm.at[s], dst_v.at[pl.ds(nj, 1)])

        def g_start(j, b):
            @pl.when(c == 0)
            def _():
                pltpu.async_copy(xa_hbm.at[src_v.at[j]], rows[b], gsem[b])

            @pl.when(c == 1)
            def _():
                pltpu.async_copy(xb_hbm.at[src_v.at[j]], rows[b], gsem[b])

        def g_wait(j, b):
            @pl.when(c == 0)
            def _():
                pltpu.make_async_copy(
                    xa_hbm.at[src_v.at[j]], rows[b], gsem[b]).wait()

            @pl.when(c == 1)
            def _():
                pltpu.make_async_copy(
                    xb_hbm.at[src_v.at[j]], rows[b], gsem[b]).wait()

        g_start(0, 0)

        def group(j2, carry):
            for b in range(2):
                j = j2 * 2 + b
                g_start(j + 1, 1 - b)
                g_wait(j, b)
                pltpu.sync_copy(rows[b], agg_sh.at[dst_v.at[j]], add=True)

                @pl.when(j % 2 == c)
                def _():
                    pltpu.sync_copy(ones_v, deg_sh.at[dst_v.at[j]],
                                    add=True)
            return carry
        lax.fori_loop(0, nj // 2, group, 0)

        g_wait(nj, 0)
        if tail:
            @pl.when(s < tail)
            def _():
                pltpu.sync_copy(rows[0], agg_sh.at[dst_v.at[nj]], add=True)

            @pl.when((s < tail) & (s % 2 == c))
            def _():
                pltpu.sync_copy(ones_v, deg_sh.at[dst_v.at[nj]], add=True)

        plsc.subcore_barrier()

        row0 = s * rows_out
        pltpu.sync_copy(agg_sh.at[pl.ds(row0, rows_out)],
                        agg_out.at[c, pl.ds(row0, rows_out)])
        pltpu.sync_copy(deg_sh.at[pl.ds(row0, rows_out)],
                        deg_out.at[c, pl.ds(row0, rows_out)])

    return agg_kernel(xa, xb, src3, dst3, tail_src, tail_dst)


def _mlp(x, agg2, deg2, W1, b1, W2, b2, W3, b3):
    n, f = x.shape
    fh = f // 2
    h1 = W1.shape[1]
    h2 = W2.shape[1]
    cc = W3.shape[1]
    bm = 2000
    grid = (n // bm,)

    def body(x_ref, a_ref, d_ref, w1_ref, b1_ref, w2_ref, b2_ref,
             w3_ref, b3_ref, o_ref):
        xb = x_ref[...]
        a = jnp.concatenate([a_ref[0], a_ref[1]], axis=-1)
        d = d_ref[0, :, 0:1] + d_ref[1, :, 0:1]
        a = a / jnp.maximum(d, 1.0)
        w1 = w1_ref[...]
        h = (jnp.dot(xb, w1[0:f], preferred_element_type=jnp.float32)
             + jnp.dot(a, w1[f:2 * f], preferred_element_type=jnp.float32)
             + b1_ref[...])
        h = jnp.maximum(h, 0.0)
        h = jnp.dot(h, w2_ref[...], preferred_element_type=jnp.float32) + b2_ref[...]
        h = jnp.maximum(h, 0.0)
        o_ref[...] = (jnp.dot(h, w3_ref[...], preferred_element_type=jnp.float32)
                      + b3_ref[...])

    return pl.pallas_call(
        body,
        grid=grid,
        in_specs=[
            pl.BlockSpec((bm, f), lambda i: (i, 0)),
            pl.BlockSpec((2, bm, fh), lambda i: (0, i, 0)),
            pl.BlockSpec((2, bm, DEGW), lambda i: (0, i, 0)),
            pl.BlockSpec((2 * f, h1), lambda i: (0, 0)),
            pl.BlockSpec((1, h1), lambda i: (0, 0)),
            pl.BlockSpec((h1, h2), lambda i: (0, 0)),
            pl.BlockSpec((1, h2), lambda i: (0, 0)),
            pl.BlockSpec((h2, cc), lambda i: (0, 0)),
            pl.BlockSpec((1, cc), lambda i: (0, 0)),
        ],
        out_specs=pl.BlockSpec((bm, cc), lambda i: (i, 0)),
        out_shape=jax.ShapeDtypeStruct((n, cc), jnp.float32),
    )(x, agg2, deg2, W1, b1, W2, b2, W3, b3)


def kernel(inputs, neighbor, W1, b1, W2, b2, W3, b3):
    x = inputs[:, 0, :]
    n, f = x.shape
    fh = f // 2
    xa = x[:, :fh]
    xb = x[:, fh:]
    src = neighbor[0]
    dst = neighbor[1]
    e = src.shape[0]

    if e % CHUNK:
        pad = CHUNK - e % CHUNK
        src = jnp.concatenate([src, jnp.zeros((pad,), jnp.int32)])
        dst = jnp.concatenate([dst, jnp.full((pad,), n, jnp.int32)])
        e += pad
    tc = e // CHUNK
    nj = (tc // NS // 2) * 2
    tail = tc - NS * nj
    if tail > NS:
        pad = (NS * (nj + 2) - tc) * CHUNK
        src = jnp.concatenate([src, jnp.zeros((pad,), jnp.int32)])
        dst = jnp.concatenate([dst, jnp.full((pad,), n, jnp.int32)])
        e += pad
        nj += 2
        tail = 0
    e_uni = NS * nj * CHUNK
    src3 = src[:e_uni].reshape(NS, nj, CHUNK)
    dst3 = dst[:e_uni].reshape(NS, nj, CHUNK)
    if tail:
        tail_src = src[e_uni:].reshape(tail, 1, CHUNK)
        tail_dst = dst[e_uni:].reshape(tail, 1, CHUNK)
    else:
        tail_src = jnp.zeros((1, 1, CHUNK), jnp.int32)
        tail_dst = jnp.full((1, 1, CHUNK), n, jnp.int32)

    n_pad = -(-(n + 1) // (NS * ZROWS)) * (NS * ZROWS)

    agg2, deg2 = _sc_aggregate(xa, xb, src3, dst3, tail_src, tail_dst,
                               n, n_pad, nj, tail)
    return _mlp(x, agg2, deg2, W1, b1.reshape(1, -1), W2, b2.reshape(1, -1),
                W3, b3.reshape(1, -1))


# speedup vs baseline: 2.0100x; 1.0321x over previous
import functools

import jax
import jax.numpy as jnp
from jax import lax
from jax.experimental import pallas as pl
from jax.experimental.pallas import tpu as pltpu
from jax.experimental.pallas import tpu_sc as plsc

NS = 16
CHUNK = 128
ZROWS = 8
DEGW = 16


def _sc_aggregate(xa, xb, src3, dst3, tail_src, tail_dst, n, n_pad, nj, tail):
    fh = xa.shape[1]
    rows_out = n_pad // NS
    zch = n_pad // (NS * ZROWS)
    npt = n_pad // NS
    nph = npt // 16

    mesh = plsc.VectorSubcoreMesh(core_axis_name="c", subcore_axis_name="s")

    @functools.partial(
        pl.kernel,
        mesh=mesh,
        compiler_params=pltpu.CompilerParams(use_tc_tiling_on_sc=False,
                                             needs_layout_passes=False),
        out_type=[
            jax.ShapeDtypeStruct((2, n_pad, fh), jnp.float32),
            jax.ShapeDtypeStruct((2, n_pad, DEGW), jnp.float32),
        ],
        scratch_types=[
            pltpu.VMEM((nj + 1, CHUNK), jnp.int32),
            pltpu.VMEM((nj + 1, CHUNK), jnp.int32),
            [pltpu.VMEM((CHUNK, fh), jnp.float32) for _ in range(2)],
            pltpu.VMEM((ZROWS, fh), jnp.float32),
            pltpu.VMEM((n_pad // 16, DEGW), jnp.float32),
            pltpu.VMEM((NS, nph, DEGW), jnp.float32),
            pltpu.VMEM_SHARED((n_pad, fh), jnp.float32),
            pltpu.VMEM_SHARED((NS, n_pad // 16, DEGW), jnp.float32),
            [pltpu.SemaphoreType.DMA for _ in range(2)],
        ],
    )
    def agg_kernel(xa_hbm, xb_hbm, src_hbm, dst_hbm, tsrc_hbm, tdst_hbm,
                   agg_out, deg_out,
                   src_v, dst_v, rows, zb_v, hist_v, rbuf, agg_sh, deg_sh,
                   gsem):
        c = lax.axis_index("c")
        s = lax.axis_index("s")
        iota16 = lax.iota(jnp.int32, 16)
        zeros16i = jnp.zeros((16,), jnp.int32)
        ones16 = jnp.ones((16,), jnp.float32)

        def fill_row(i, carry):
            for k in range(fh // 16):
                zb_v[i, pl.ds(k * 16, 16)] = jnp.zeros((16,), jnp.float32)
            return carry
        lax.fori_loop(0, ZROWS, fill_row, 0)

        def zero_hist(i, carry):
            hist_v[i, :] = jnp.zeros((16,), jnp.float32)
            return carry
        lax.fori_loop(0, n_pad // 16, zero_hist, 0)

        def zero_chunk(t, carry):
            row0 = s * (zch * ZROWS) + t * ZROWS
            pltpu.sync_copy(zb_v, agg_sh.at[pl.ds(row0, ZROWS)])
            return carry
        lax.fori_loop(0, zch, zero_chunk, 0)

        plsc.subcore_barrier()

        pltpu.sync_copy(src_hbm.at[s], src_v.at[pl.ds(0, nj)])
        pltpu.sync_copy(dst_hbm.at[s], dst_v.at[pl.ds(0, nj)])
        for k in range(CHUNK // 16):
            src_v[nj, pl.ds(k * 16, 16)] = jnp.zeros((16,), jnp.int32)
            dst_v[nj, pl.ds(k * 16, 16)] = jnp.full((16,), n, jnp.int32)
        if tail:
            @pl.when(s < tail)
            def _():
                pltpu.sync_copy(tsrc_hbm.at[s], src_v.at[pl.ds(nj, 1)])
                pltpu.sync_copy(tdst_hbm.at[s], dst_v.at[pl.ds(nj, 1)])

        def g_start(j, b):
            @pl.when(c == 0)
            def _():
                pltpu.async_copy(xa_hbm.at[src_v.at[j]], rows[b], gsem[b])

            @pl.when(c == 1)
            def _():
                pltpu.async_copy(xb_hbm.at[src_v.at[j]], rows[b], gsem[b])

        def g_wait(j, b):
            @pl.when(c == 0)
            def _():
                pltpu.make_async_copy(
                    xa_hbm.at[src_v.at[j]], rows[b], gsem[b]).wait()

            @pl.when(c == 1)
            def _():
                pltpu.make_async_copy(
                    xb_hbm.at[src_v.at[j]], rows[b], gsem[b]).wait()

        g_start(0, 0)

        def hist_chunk(j):
            for k in range(CHUNK // 16):
                v = dst_v[j, pl.ds(k * 16, 16)]
                r = lax.shift_right_logical(v, 4)
                c2 = lax.bitwise_and(v, 15)
                plsc.addupdate_scatter(hist_v, [r, c2], ones16)

        def group(j2, carry):
            for b in range(2):
                j = j2 * 2 + b
                g_start(j + 1, 1 - b)

                @pl.when(j % 2 == c)
                def _():
                    hist_chunk(j)

                g_wait(j, b)
                pltpu.sync_copy(rows[b], agg_sh.at[dst_v.at[j]], add=True)
            return carry
        lax.fori_loop(0, nj // 2, group, 0)

        g_wait(nj, 0)
        if tail:
            @pl.when(s < tail)
            def _():
                pltpu.sync_copy(rows[0], agg_sh.at[dst_v.at[nj]], add=True)

            @pl.when((s < tail) & ((nj % 2) == c))
            def _():
                hist_chunk(nj)

        pltpu.sync_copy(hist_v, deg_sh.at[s])

        plsc.subcore_barrier()

        for i in range(NS):
            pltpu.sync_copy(deg_sh.at[i, pl.ds(s * nph, nph)], rbuf.at[i])

        def red(t, carry):
            acc = rbuf[0, t, :]
            for i in range(1, NS):
                acc = acc + rbuf[i, t, :]
            plsc.store_scatter(hist_v, [t * 16 + iota16, zeros16i], acc)
            return carry
        lax.fori_loop(0, nph, red, 0)

        row0 = s * rows_out
        pltpu.sync_copy(agg_sh.at[pl.ds(row0, rows_out)],
                        agg_out.at[c, pl.ds(row0, rows_out)])
        pltpu.sync_copy(hist_v.at[pl.ds(0, npt)],
                        deg_out.at[c, pl.ds(s * npt, npt)])

    return agg_kernel(xa, xb, src3, dst3, tail_src, tail_dst)


def _mlp(x, agg2, deg2, W1, b1, W2, b2, W3, b3):
    n, f = x.shape
    fh = f // 2
    h1 = W1.shape[1]
    h2 = W2.shape[1]
    cc = W3.shape[1]
    bm = 2000
    grid = (n // bm,)

    def body(x_ref, a_ref, d_ref, w1_ref, b1_ref, w2_ref, b2_ref,
             w3_ref, b3_ref, o_ref):
        xb = x_ref[...]
        a = jnp.concatenate([a_ref[0], a_ref[1]], axis=-1)
        d = d_ref[0, :, 0:1] + d_ref[1, :, 0:1]
        a = a / jnp.maximum(d, 1.0)
        w1 = w1_ref[...]
        h = (jnp.dot(xb, w1[0:f], preferred_element_type=jnp.float32)
             + jnp.dot(a, w1[f:2 * f], preferred_element_type=jnp.float32)
             + b1_ref[...])
        h = jnp.maximum(h, 0.0)
        h = jnp.dot(h, w2_ref[...], preferred_element_type=jnp.float32) + b2_ref[...]
        h = jnp.maximum(h, 0.0)
        o_ref[...] = (jnp.dot(h, w3_ref[...], preferred_element_type=jnp.float32)
                      + b3_ref[...])

    return pl.pallas_call(
        body,
        grid=grid,
        in_specs=[
            pl.BlockSpec((bm, f), lambda i: (i, 0)),
            pl.BlockSpec((2, bm, fh), lambda i: (0, i, 0)),
            pl.BlockSpec((2, bm, DEGW), lambda i: (0, i, 0)),
            pl.BlockSpec((2 * f, h1), lambda i: (0, 0)),
            pl.BlockSpec((1, h1), lambda i: (0, 0)),
            pl.BlockSpec((h1, h2), lambda i: (0, 0)),
            pl.BlockSpec((1, h2), lambda i: (0, 0)),
            pl.BlockSpec((h2, cc), lambda i: (0, 0)),
            pl.BlockSpec((1, cc), lambda i: (0, 0)),
        ],
        out_specs=pl.BlockSpec((bm, cc), lambda i: (i, 0)),
        out_shape=jax.ShapeDtypeStruct((n, cc), jnp.float32),
    )(x, agg2, deg2, W1, b1, W2, b2, W3, b3)


def kernel(inputs, neighbor, W1, b1, W2, b2, W3, b3):
    x = inputs[:, 0, :]
    n, f = x.shape
    fh = f // 2
    xa = x[:, :fh]
    xb = x[:, fh:]
    src = neighbor[0]
    dst = neighbor[1]
    e = src.shape[0]

    if e % CHUNK:
        pad = CHUNK - e % CHUNK
        src = jnp.concatenate([src, jnp.zeros((pad,), jnp.int32)])
        dst = jnp.concatenate([dst, jnp.full((pad,), n, jnp.int32)])
        e += pad
    tc = e // CHUNK
    nj = (tc // NS // 2) * 2
    tail = tc - NS * nj
    if tail > NS:
        pad = (NS * (nj + 2) - tc) * CHUNK
        src = jnp.concatenate([src, jnp.zeros((pad,), jnp.int32)])
        dst = jnp.concatenate([dst, jnp.full((pad,), n, jnp.int32)])
        e += pad
        nj += 2
        tail = 0
    e_uni = NS * nj * CHUNK
    src3 = src[:e_uni].reshape(NS, nj, CHUNK)
    dst3 = dst[:e_uni].reshape(NS, nj, CHUNK)
    if tail:
        tail_src = src[e_uni:].reshape(tail, 1, CHUNK)
        tail_dst = dst[e_uni:].reshape(tail, 1, CHUNK)
    else:
        tail_src = jnp.zeros((1, 1, CHUNK), jnp.int32)
        tail_dst = jnp.full((1, 1, CHUNK), n, jnp.int32)

    m = max(NS * ZROWS, NS * 16)
    n_pad = -(-(n + 1) // m) * m

    agg2, deg2 = _sc_aggregate(xa, xb, src3, dst3, tail_src, tail_dst,
                               n, n_pad, nj, tail)
    return _mlp(x, agg2, deg2, W1, b1.reshape(1, -1), W2, b2.reshape(1, -1),
                W3, b3.reshape(1, -1))


# speedup vs baseline: 2.0335x; 1.0117x over previous
import functools

import jax
import jax.numpy as jnp
from jax import lax
from jax.experimental import pallas as pl
from jax.experimental.pallas import tpu as pltpu
from jax.experimental.pallas import tpu_sc as plsc

NS = 16
CHUNK = 128
ZROWS = 32
DEGW = 16


def _sc_aggregate(xa, xb, src3, dst3, tail_src, tail_dst, n, n_pad, nj, tail):
    fh = xa.shape[1]
    rows_out = n_pad // NS
    zch = n_pad // (NS * ZROWS)
    npt = n_pad // NS
    nph = npt // 16

    mesh = plsc.VectorSubcoreMesh(core_axis_name="c", subcore_axis_name="s")

    @functools.partial(
        pl.kernel,
        mesh=mesh,
        compiler_params=pltpu.CompilerParams(use_tc_tiling_on_sc=False,
                                             needs_layout_passes=False),
        out_type=[
            jax.ShapeDtypeStruct((2, n_pad, fh), jnp.float32),
            jax.ShapeDtypeStruct((2, n_pad, DEGW), jnp.float32),
        ],
        scratch_types=[
            pltpu.VMEM((nj + 1, CHUNK), jnp.int32),
            pltpu.VMEM((nj + 1, CHUNK), jnp.int32),
            [pltpu.VMEM((CHUNK, fh), jnp.float32) for _ in range(2)],
            pltpu.VMEM((ZROWS, fh), jnp.float32),
            pltpu.VMEM((n_pad // 16, DEGW), jnp.float32),
            pltpu.VMEM((NS, nph, DEGW), jnp.float32),
            pltpu.VMEM_SHARED((n_pad, fh), jnp.float32),
            pltpu.VMEM_SHARED((NS, n_pad // 16, DEGW), jnp.float32),
            [pltpu.SemaphoreType.DMA for _ in range(2)],
        ],
    )
    def agg_kernel(xa_hbm, xb_hbm, src_hbm, dst_hbm, tsrc_hbm, tdst_hbm,
                   agg_out, deg_out,
                   src_v, dst_v, rows, zb_v, hist_v, rbuf, agg_sh, deg_sh,
                   gsem):
        c = lax.axis_index("c")
        s = lax.axis_index("s")
        iota16 = lax.iota(jnp.int32, 16)
        zeros16i = jnp.zeros((16,), jnp.int32)
        ones16 = jnp.ones((16,), jnp.float32)

        pltpu.sync_copy(src_hbm.at[s], src_v.at[pl.ds(0, nj)])
        pltpu.sync_copy(dst_hbm.at[s], dst_v.at[pl.ds(0, nj)])
        for k in range(CHUNK // 16):
            src_v[nj, pl.ds(k * 16, 16)] = jnp.zeros((16,), jnp.int32)
            dst_v[nj, pl.ds(k * 16, 16)] = jnp.full((16,), n, jnp.int32)
        if tail:
            @pl.when(s < tail)
            def _():
                pltpu.sync_copy(tsrc_hbm.at[s], src_v.at[pl.ds(nj, 1)])
                pltpu.sync_copy(tdst_hbm.at[s], dst_v.at[pl.ds(nj, 1)])

        def g_start(j, b):
            @pl.when(c == 0)
            def _():
                pltpu.async_copy(xa_hbm.at[src_v.at[j]], rows[b], gsem[b])

            @pl.when(c == 1)
            def _():
                pltpu.async_copy(xb_hbm.at[src_v.at[j]], rows[b], gsem[b])

        def g_wait(j, b):
            @pl.when(c == 0)
            def _():
                pltpu.make_async_copy(
                    xa_hbm.at[src_v.at[j]], rows[b], gsem[b]).wait()

            @pl.when(c == 1)
            def _():
                pltpu.make_async_copy(
                    xb_hbm.at[src_v.at[j]], rows[b], gsem[b]).wait()

        g_start(0, 0)

        def fill_row(i, carry):
            for k in range(fh // 16):
                zb_v[i, pl.ds(k * 16, 16)] = jnp.zeros((16,), jnp.float32)
            return carry
        lax.fori_loop(0, ZROWS, fill_row, 0)

        def zero_hist(i, carry):
            hist_v[i, :] = jnp.zeros((16,), jnp.float32)
            return carry
        lax.fori_loop(0, n_pad // 16, zero_hist, 0)

        def zero_chunk(t, carry):
            row0 = s * (zch * ZROWS) + t * ZROWS
            pltpu.sync_copy(zb_v, agg_sh.at[pl.ds(row0, ZROWS)])
            return carry
        lax.fori_loop(0, zch, zero_chunk, 0)

        plsc.subcore_barrier()


        def hist_chunk(j):
            for k in range(CHUNK // 16):
                v = dst_v[j, pl.ds(k * 16, 16)]
                r = lax.shift_right_logical(v, 4)
                c2 = lax.bitwise_and(v, 15)
                plsc.addupdate_scatter(hist_v, [r, c2], ones16)

        def group(j2, carry):
            for b in range(2):
                j = j2 * 2 + b
                g_start(j + 1, 1 - b)

                @pl.when(j % 2 == c)
                def _():
                    hist_chunk(j)

                g_wait(j, b)
                pltpu.sync_copy(rows[b], agg_sh.at[dst_v.at[j]], add=True)
            return carry
        lax.fori_loop(0, nj // 2, group, 0)

        g_wait(nj, 0)
        if tail:
            @pl.when(s < tail)
            def _():
                pltpu.sync_copy(rows[0], agg_sh.at[dst_v.at[nj]], add=True)

            @pl.when((s < tail) & ((nj % 2) == c))
            def _():
                hist_chunk(nj)

        pltpu.sync_copy(hist_v, deg_sh.at[s])

        plsc.subcore_barrier()

        for i in range(NS):
            pltpu.sync_copy(deg_sh.at[i, pl.ds(s * nph, nph)], rbuf.at[i])

        def red(t, carry):
            acc = rbuf[0, t, :]
            for i in range(1, NS):
                acc = acc + rbuf[i, t, :]
            plsc.store_scatter(hist_v, [t * 16 + iota16, zeros16i], acc)
            return carry
        lax.fori_loop(0, nph, red, 0)

        row0 = s * rows_out
        pltpu.sync_copy(agg_sh.at[pl.ds(row0, rows_out)],
                        agg_out.at[c, pl.ds(row0, rows_out)])
        pltpu.sync_copy(hist_v.at[pl.ds(0, npt)],
                        deg_out.at[c, pl.ds(s * npt, npt)])

    return agg_kernel(xa, xb, src3, dst3, tail_src, tail_dst)


def _mlp(x, agg2, deg2, W1, b1, W2, b2, W3, b3):
    n, f = x.shape
    fh = f // 2
    h1 = W1.shape[1]
    h2 = W2.shape[1]
    cc = W3.shape[1]
    bm = 2000
    grid = (n // bm,)

    def body(x_ref, a_ref, d_ref, w1_ref, b1_ref, w2_ref, b2_ref,
             w3_ref, b3_ref, o_ref):
        xb = x_ref[...]
        a = jnp.concatenate([a_ref[0], a_ref[1]], axis=-1)
        d = d_ref[0, :, 0:1] + d_ref[1, :, 0:1]
        a = a / jnp.maximum(d, 1.0)
        w1 = w1_ref[...]
        h = (jnp.dot(xb, w1[0:f], preferred_element_type=jnp.float32)
             + jnp.dot(a, w1[f:2 * f], preferred_element_type=jnp.float32)
             + b1_ref[...])
        h = jnp.maximum(h, 0.0)
        h = jnp.dot(h, w2_ref[...], preferred_element_type=jnp.float32) + b2_ref[...]
        h = jnp.maximum(h, 0.0)
        o_ref[...] = (jnp.dot(h, w3_ref[...], preferred_element_type=jnp.float32)
                      + b3_ref[...])

    return pl.pallas_call(
        body,
        grid=grid,
        in_specs=[
            pl.BlockSpec((bm, f), lambda i: (i, 0)),
            pl.BlockSpec((2, bm, fh), lambda i: (0, i, 0)),
            pl.BlockSpec((2, bm, DEGW), lambda i: (0, i, 0)),
            pl.BlockSpec((2 * f, h1), lambda i: (0, 0)),
            pl.BlockSpec((1, h1), lambda i: (0, 0)),
            pl.BlockSpec((h1, h2), lambda i: (0, 0)),
            pl.BlockSpec((1, h2), lambda i: (0, 0)),
            pl.BlockSpec((h2, cc), lambda i: (0, 0)),
            pl.BlockSpec((1, cc), lambda i: (0, 0)),
        ],
        out_specs=pl.BlockSpec((bm, cc), lambda i: (i, 0)),
        out_shape=jax.ShapeDtypeStruct((n, cc), jnp.float32),
    )(x, agg2, deg2, W1, b1, W2, b2, W3, b3)


def kernel(inputs, neighbor, W1, b1, W2, b2, W3, b3):
    x = inputs[:, 0, :]
    n, f = x.shape
    fh = f // 2
    xa = x[:, :fh]
    xb = x[:, fh:]
    src = neighbor[0]
    dst = neighbor[1]
    e = src.shape[0]

    if e % CHUNK:
        pad = CHUNK - e % CHUNK
        src = jnp.concatenate([src, jnp.zeros((pad,), jnp.int32)])
        dst = jnp.concatenate([dst, jnp.full((pad,), n, jnp.int32)])
        e += pad
    tc = e // CHUNK
    nj = (tc // NS // 2) * 2
    tail = tc - NS * nj
    if tail > NS:
        pad = (NS * (nj + 2) - tc) * CHUNK
        src = jnp.concatenate([src, jnp.zeros((pad,), jnp.int32)])
        dst = jnp.concatenate([dst, jnp.full((pad,), n, jnp.int32)])
        e += pad
        nj += 2
        tail = 0
    e_uni = NS * nj * CHUNK
    src3 = src[:e_uni].reshape(NS, nj, CHUNK)
    dst3 = dst[:e_uni].reshape(NS, nj, CHUNK)
    if tail:
        tail_src = src[e_uni:].reshape(tail, 1, CHUNK)
        tail_dst = dst[e_uni:].reshape(tail, 1, CHUNK)
    else:
        tail_src = jnp.zeros((1, 1, CHUNK), jnp.int32)
        tail_dst = jnp.full((1, 1, CHUNK), n, jnp.int32)

    m = max(NS * ZROWS, NS * 16)
    n_pad = -(-(n + 1) // m) * m

    agg2, deg2 = _sc_aggregate(xa, xb, src3, dst3, tail_src, tail_dst,
                               n, n_pad, nj, tail)
    return _mlp(x, agg2, deg2, W1, b1.reshape(1, -1), W2, b2.reshape(1, -1),
                W3, b3.reshape(1, -1))


# speedup vs baseline: 2.0451x; 1.0057x over previous
import functools

import jax
import jax.numpy as jnp
from jax import lax
from jax.experimental import pallas as pl
from jax.experimental.pallas import tpu as pltpu
from jax.experimental.pallas import tpu_sc as plsc

NS = 16
CHUNK = 128
ZROWS = 32
DEGW = 16


def _sc_aggregate(xa, xb, src3, dst3, tail_src, tail_dst, n, n_pad, nj, tail):
    fh = xa.shape[1]
    rows_out = n_pad // NS
    zch = n_pad // (NS * ZROWS)
    npt = n_pad // NS
    nph = npt // 16

    mesh = plsc.VectorSubcoreMesh(core_axis_name="c", subcore_axis_name="s")

    @functools.partial(
        pl.kernel,
        mesh=mesh,
        compiler_params=pltpu.CompilerParams(use_tc_tiling_on_sc=False,
                                             needs_layout_passes=False),
        out_type=[
            jax.ShapeDtypeStruct((2, n_pad, fh), jnp.float32),
            jax.ShapeDtypeStruct((2, n_pad, DEGW), jnp.float32),
        ],
        scratch_types=[
            pltpu.VMEM((nj + 1, CHUNK), jnp.int32),
            pltpu.VMEM((nj + 1, CHUNK), jnp.int32),
            [pltpu.VMEM((CHUNK, fh), jnp.float32) for _ in range(2)],
            pltpu.VMEM((ZROWS, fh), jnp.float32),
            pltpu.VMEM((n_pad // 16, DEGW), jnp.float32),
            pltpu.VMEM((NS, nph, DEGW), jnp.float32),
            pltpu.VMEM_SHARED((n_pad, fh), jnp.float32),
            pltpu.VMEM_SHARED((NS, n_pad // 16, DEGW), jnp.float32),
            [pltpu.SemaphoreType.DMA for _ in range(2)],
            pltpu.SemaphoreType.DMA,
        ],
    )
    def agg_kernel(xa_hbm, xb_hbm, src_hbm, dst_hbm, tsrc_hbm, tdst_hbm,
                   agg_out, deg_out,
                   src_v, dst_v, rows, zb_v, hist_v, rbuf, agg_sh, deg_sh,
                   gsem, zsem):
        c = lax.axis_index("c")
        s = lax.axis_index("s")
        iota16 = lax.iota(jnp.int32, 16)
        zeros16i = jnp.zeros((16,), jnp.int32)
        ones16 = jnp.ones((16,), jnp.float32)

        pltpu.sync_copy(src_hbm.at[s], src_v.at[pl.ds(0, nj)])
        pltpu.sync_copy(dst_hbm.at[s], dst_v.at[pl.ds(0, nj)])
        for k in range(CHUNK // 16):
            src_v[nj, pl.ds(k * 16, 16)] = jnp.zeros((16,), jnp.int32)
            dst_v[nj, pl.ds(k * 16, 16)] = jnp.full((16,), n, jnp.int32)
        if tail:
            @pl.when(s < tail)
            def _():
                pltpu.sync_copy(tsrc_hbm.at[s], src_v.at[pl.ds(nj, 1)])
                pltpu.sync_copy(tdst_hbm.at[s], dst_v.at[pl.ds(nj, 1)])

        def g_start(j, b):
            @pl.when(c == 0)
            def _():
                pltpu.async_copy(xa_hbm.at[src_v.at[j]], rows[b], gsem[b])

            @pl.when(c == 1)
            def _():
                pltpu.async_copy(xb_hbm.at[src_v.at[j]], rows[b], gsem[b])

        def g_wait(j, b):
            @pl.when(c == 0)
            def _():
                pltpu.make_async_copy(
                    xa_hbm.at[src_v.at[j]], rows[b], gsem[b]).wait()

            @pl.when(c == 1)
            def _():
                pltpu.make_async_copy(
                    xb_hbm.at[src_v.at[j]], rows[b], gsem[b]).wait()

        g_start(0, 0)

        def fill_row(i, carry):
            for k in range(fh // 16):
                zb_v[i, pl.ds(k * 16, 16)] = jnp.zeros((16,), jnp.float32)
            return carry
        lax.fori_loop(0, ZROWS, fill_row, 0)

        def zero_hist(i, carry):
            hist_v[i, :] = jnp.zeros((16,), jnp.float32)
            return carry
        lax.fori_loop(0, n_pad // 16, zero_hist, 0)

        def zero_chunk(t, carry):
            row0 = s * (zch * ZROWS) + t * ZROWS
            pltpu.async_copy(zb_v, agg_sh.at[pl.ds(row0, ZROWS)], zsem)
            return carry
        lax.fori_loop(0, zch, zero_chunk, 0)

        def zero_drain(t, carry):
            pltpu.make_async_copy(zb_v, agg_sh.at[pl.ds(0, ZROWS)],
                                  zsem).wait()
            return carry
        lax.fori_loop(0, zch, zero_drain, 0)

        plsc.subcore_barrier()


        def hist_chunk(j):
            for k in range(CHUNK // 16):
                v = dst_v[j, pl.ds(k * 16, 16)]
                r = lax.shift_right_logical(v, 4)
                c2 = lax.bitwise_and(v, 15)
                plsc.addupdate_scatter(hist_v, [r, c2], ones16)

        def group(j2, carry):
            for b in range(2):
                j = j2 * 2 + b
                g_start(j + 1, 1 - b)

                @pl.when(j % 2 == c)
                def _():
                    hist_chunk(j)

                g_wait(j, b)
                pltpu.sync_copy(rows[b], agg_sh.at[dst_v.at[j]], add=True)
            return carry
        lax.fori_loop(0, nj // 2, group, 0)

        g_wait(nj, 0)
        if tail:
            @pl.when(s < tail)
            def _():
                pltpu.sync_copy(rows[0], agg_sh.at[dst_v.at[nj]], add=True)

            @pl.when((s < tail) & ((nj % 2) == c))
            def _():
                hist_chunk(nj)

        pltpu.sync_copy(hist_v, deg_sh.at[s])

        plsc.subcore_barrier()

        for i in range(NS):
            pltpu.sync_copy(deg_sh.at[i, pl.ds(s * nph, nph)], rbuf.at[i])

        def red(t, carry):
            acc = rbuf[0, t, :]
            for i in range(1, NS):
                acc = acc + rbuf[i, t, :]
            plsc.store_scatter(hist_v, [t * 16 + iota16, zeros16i], acc)
            return carry
        lax.fori_loop(0, nph, red, 0)

        row0 = s * rows_out
        pltpu.sync_copy(agg_sh.at[pl.ds(row0, rows_out)],
                        agg_out.at[c, pl.ds(row0, rows_out)])
        pltpu.sync_copy(hist_v.at[pl.ds(0, npt)],
                        deg_out.at[c, pl.ds(s * npt, npt)])

    return agg_kernel(xa, xb, src3, dst3, tail_src, tail_dst)


def _mlp(x, agg2, deg2, W1, b1, W2, b2, W3, b3):
    n, f = x.shape
    fh = f // 2
    h1 = W1.shape[1]
    h2 = W2.shape[1]
    cc = W3.shape[1]
    bm = 2000
    grid = (n // bm,)

    def body(x_ref, a_ref, d_ref, w1_ref, b1_ref, w2_ref, b2_ref,
             w3_ref, b3_ref, o_ref):
        xb = x_ref[...]
        a = jnp.concatenate([a_ref[0], a_ref[1]], axis=-1)
        d = d_ref[0, :, 0:1] + d_ref[1, :, 0:1]
        a = a / jnp.maximum(d, 1.0)
        w1 = w1_ref[...]
        h = (jnp.dot(xb, w1[0:f], preferred_element_type=jnp.float32)
             + jnp.dot(a, w1[f:2 * f], preferred_element_type=jnp.float32)
             + b1_ref[...])
        h = jnp.maximum(h, 0.0)
        h = jnp.dot(h, w2_ref[...], preferred_element_type=jnp.float32) + b2_ref[...]
        h = jnp.maximum(h, 0.0)
        o_ref[...] = (jnp.dot(h, w3_ref[...], preferred_element_type=jnp.float32)
                      + b3_ref[...])

    return pl.pallas_call(
        body,
        grid=grid,
        in_specs=[
            pl.BlockSpec((bm, f), lambda i: (i, 0)),
            pl.BlockSpec((2, bm, fh), lambda i: (0, i, 0)),
            pl.BlockSpec((2, bm, DEGW), lambda i: (0, i, 0)),
            pl.BlockSpec((2 * f, h1), lambda i: (0, 0)),
            pl.BlockSpec((1, h1), lambda i: (0, 0)),
            pl.BlockSpec((h1, h2), lambda i: (0, 0)),
            pl.BlockSpec((1, h2), lambda i: (0, 0)),
            pl.BlockSpec((h2, cc), lambda i: (0, 0)),
            pl.BlockSpec((1, cc), lambda i: (0, 0)),
        ],
        out_specs=pl.BlockSpec((bm, cc), lambda i: (i, 0)),
        out_shape=jax.ShapeDtypeStruct((n, cc), jnp.float32),
    )(x, agg2, deg2, W1, b1, W2, b2, W3, b3)


def kernel(inputs, neighbor, W1, b1, W2, b2, W3, b3):
    x = inputs[:, 0, :]
    n, f = x.shape
    fh = f // 2
    xa = x[:, :fh]
    xb = x[:, fh:]
    src = neighbor[0]
    dst = neighbor[1]
    e = src.shape[0]

    if e % CHUNK:
        pad = CHUNK - e % CHUNK
        src = jnp.concatenate([src, jnp.zeros((pad,), jnp.int32)])
        dst = jnp.concatenate([dst, jnp.full((pad,), n, jnp.int32)])
        e += pad
    tc = e // CHUNK
    nj = (tc // NS // 2) * 2
    tail = tc - NS * nj
    if tail > NS:
        pad = (NS * (nj + 2) - tc) * CHUNK
        src = jnp.concatenate([src, jnp.zeros((pad,), jnp.int32)])
        dst = jnp.concatenate([dst, jnp.full((pad,), n, jnp.int32)])
        e += pad
        nj += 2
        tail = 0
    e_uni = NS * nj * CHUNK
    src3 = src[:e_uni].reshape(NS, nj, CHUNK)
    dst3 = dst[:e_uni].reshape(NS, nj, CHUNK)
    if tail:
        tail_src = src[e_uni:].reshape(tail, 1, CHUNK)
        tail_dst = dst[e_uni:].reshape(tail, 1, CHUNK)
    else:
        tail_src = jnp.zeros((1, 1, CHUNK), jnp.int32)
        tail_dst = jnp.full((1, 1, CHUNK), n, jnp.int32)

    m = max(NS * ZROWS, NS * 16)
    n_pad = -(-(n + 1) // m) * m

    agg2, deg2 = _sc_aggregate(xa, xb, src3, dst3, tail_src, tail_dst,
                               n, n_pad, nj, tail)
    return _mlp(x, agg2, deg2, W1, b1.reshape(1, -1), W2, b2.reshape(1, -1),
                W3, b3.reshape(1, -1))
